# Initial kernel scaffold; baseline (speedup 1.0000x reference)
import jax
import jax.numpy as jnp
from jax.experimental import pallas as pl


def kernel(x, table, W1, b1, W2, b2):
    raise NotImplementedError("write your pallas kernel here")


# speedup vs baseline: 1.3398x; 1.3398x over previous
import functools

import jax
import jax.numpy as jnp
from jax import lax
from jax.experimental import pallas as pl
from jax.experimental.pallas import tpu as pltpu
from jax.experimental.pallas import tpu_sc as plsc

ITEMS = 1000000
D = 64
L = 200
B = 4096
MAXN = 1.0

NC = 2
NS = 16
NW = NC * NS
BPW = B // NW
HALF = L // 2
UNROLL = 4
W_STEP = 0.9 / (L - 1)


def _rsqrt_vec(ss):
    i = lax.bitcast_convert_type(ss, jnp.int32)
    i = jnp.int32(0x5F3759DF) - (i >> 1)
    y = lax.bitcast_convert_type(i, jnp.float32)
    h = jnp.float32(0.5) * ss
    y = y * (jnp.float32(1.5) - (h * y) * y)
    y = y * (jnp.float32(1.5) - (h * y) * y)
    y = y * (jnp.float32(1.5) - (h * y) * y)
    return y


_GATHER_DNUMS = lax.GatherDimensionNumbers(
    offset_dims=(), collapsed_slice_dims=(0,), start_index_map=(0,)
)


def _shuffle(q, idx2):
    return lax.gather(
        q, idx2, _GATHER_DNUMS, slice_sizes=(1,),
        mode=lax.GatherScatterMode.PROMISE_IN_BOUNDS,
    )


def _lane_sum(q, perms):
    for idx2 in perms:
        q = q + _shuffle(q, idx2)
    return q


def _pool_sc(x3, table):
    mesh = plsc.VectorSubcoreMesh(
        core_axis_name="c", subcore_axis_name="s", num_cores=NC, num_subcores=NS
    )

    @functools.partial(
        pl.kernel,
        out_type=jax.ShapeDtypeStruct((B, D), jnp.float32),
        mesh=mesh,
        compiler_params=pltpu.CompilerParams(use_tc_tiling_on_sc=False),
        scratch_types=[
            pltpu.VMEM((BPW, 2, HALF), jnp.int32),
            pltpu.VMEM((2, HALF, D), jnp.float32),
            pltpu.VMEM((BPW, D), jnp.float32),
            pltpu.SemaphoreType.DMA,
            pltpu.SemaphoreType.DMA,
        ],
    )
    def pool(x_hbm, tab_hbm, out_hbm, idx_v, rows_v, out_v, sem0, sem1):
        wid = lax.axis_index("s") * NC + lax.axis_index("c")
        base = wid * BPW
        sems = (sem0, sem1)

        pltpu.sync_copy(x_hbm.at[pl.ds(base, BPW)], idx_v)

        def gather_start(b, j, p):
            pltpu.make_async_copy(
                tab_hbm.at[idx_v.at[b, j]], rows_v.at[p], sems[p]
            ).start()

        def gather_wait(b, j, p):
            pltpu.make_async_copy(
                tab_hbm.at[idx_v.at[b, j]], rows_v.at[p], sems[p]
            ).wait()

        lanes = lax.iota(jnp.int32, 16)
        perms = tuple((lanes ^ k)[:, None] for k in (8, 4, 2, 1))

        def compute_half(p, j, acc):
            def step(it, acc):
                for u in range(UNROLL):
                    l = it * UNROLL + u
                    a0, a1, a2, a3 = acc
                    r0 = rows_v[p, l, pl.ds(0, 16)]
                    r1 = rows_v[p, l, pl.ds(16, 16)]
                    r2 = rows_v[p, l, pl.ds(32, 16)]
                    r3 = rows_v[p, l, pl.ds(48, 16)]
                    q = r0 * r0 + r1 * r1 + r2 * r2 + r3 * r3
                    ss = _lane_sum(q, perms)
                    y = _rsqrt_vec(ss)
                    n = ss * y
                    factor = jnp.where(n > MAXN, y, jnp.float32(1.0))
                    wl = jnp.float32(0.1) + (
                        jnp.float32(j * HALF) + l.astype(jnp.float32)
                    ) * jnp.float32(W_STEP)
                    cf = factor * wl
                    acc = (a0 + cf * r0, a1 + cf * r1, a2 + cf * r2, a3 + cf * r3)
                return acc

            return lax.fori_loop(0, HALF // UNROLL, step, acc, unroll=False)

        def seq_body(b, _):
            gather_start(b, 1, 1)
            gather_wait(b, 0, 0)
            zero = jnp.zeros((16,), jnp.float32)
            acc = compute_half(0, 0, (zero, zero, zero, zero))

            @pl.when(b + 1 < BPW)
            def _():
                gather_start(b + 1, 0, 0)

            gather_wait(b, 1, 1)
            a0, a1, a2, a3 = compute_half(1, 1, acc)
            out_v[b, pl.ds(0, 16)] = a0
            out_v[b, pl.ds(16, 16)] = a1
            out_v[b, pl.ds(32, 16)] = a2
            out_v[b, pl.ds(48, 16)] = a3
            return 0

        gather_start(0, 0, 0)
        lax.fori_loop(0, BPW, seq_body, 0, unroll=False)
        pltpu.sync_copy(out_v, out_hbm.at[pl.ds(base, BPW)])

    return pool(x3, table)


def _mlp_tc(s_raw, W1, b1, W2, b2):
    def body(s_ref, w1_ref, b1_ref, w2_ref, b2_ref, o_ref):
        s = s_ref[...]
        ss = jnp.sum(s * s, axis=-1, keepdims=True)
        s = s * jax.lax.rsqrt(jnp.maximum(ss, jnp.float32(1e-24)))
        h = jnp.tanh(
            lax.dot_general(
                s, w1_ref[...], (((1,), (1,)), ((), ())),
                preferred_element_type=jnp.float32,
            )
            + b1_ref[...]
        )
        o = jnp.tanh(
            lax.dot_general(
                h, w2_ref[...], (((1,), (1,)), ((), ())),
                preferred_element_type=jnp.float32,
            )
            + b2_ref[...]
        )
        o_ref[...] = o

    return pl.pallas_call(
        body,
        out_shape=jax.ShapeDtypeStruct((B, D), jnp.float32),
    )(s_raw, W1, b1, W2, b2)


def kernel(x, table, W1, b1, W2, b2):
    x3 = x.astype(jnp.int32).reshape(B, 2, HALF)
    s_raw = _pool_sc(x3, table)
    return _mlp_tc(s_raw, W1, b1, W2, b2)


# speedup vs baseline: 1.4297x; 1.0671x over previous
import functools

import jax
import jax.numpy as jnp
from jax import lax
from jax.experimental import pallas as pl
from jax.experimental.pallas import tpu as pltpu
from jax.experimental.pallas import tpu_sc as plsc

ITEMS = 1000000
D = 64
L = 200
B = 4096
MAXN = 1.0

NC = 2
NS = 16
NW = NC * NS
BPW = B // NW
HALF = L // 2
GROUP = 10
W_STEP = 0.9 / (L - 1)


def _rsqrt_vec(ss):
    i = lax.bitcast_convert_type(ss, jnp.int32)
    i = jnp.int32(0x5F3759DF) - (i >> 1)
    y = lax.bitcast_convert_type(i, jnp.float32)
    h = jnp.float32(0.5) * ss
    y = y * (jnp.float32(1.5) - (h * y) * y)
    y = y * (jnp.float32(1.5) - (h * y) * y)
    y = y * (jnp.float32(1.5) - (h * y) * y)
    return y


_GATHER_DNUMS = lax.GatherDimensionNumbers(
    offset_dims=(), collapsed_slice_dims=(0,), start_index_map=(0,)
)


def _shuffle(q, idx2):
    return lax.gather(
        q, idx2, _GATHER_DNUMS, slice_sizes=(1,),
        mode=lax.GatherScatterMode.PROMISE_IN_BOUNDS,
    )


def _lane_sum(q, perms):
    for idx2 in perms:
        q = q + _shuffle(q, idx2)
    return q


def _pool_sc(x3, table):
    mesh = plsc.VectorSubcoreMesh(
        core_axis_name="c", subcore_axis_name="s", num_cores=NC, num_subcores=NS
    )

    @functools.partial(
        pl.kernel,
        out_type=jax.ShapeDtypeStruct((B, D), jnp.float32),
        mesh=mesh,
        compiler_params=pltpu.CompilerParams(use_tc_tiling_on_sc=False),
        scratch_types=[
            pltpu.VMEM((BPW, 2, HALF), jnp.int32),
            pltpu.VMEM((2, HALF, D), jnp.float32),
            pltpu.VMEM((BPW, D), jnp.float32),
            pltpu.SemaphoreType.DMA,
            pltpu.SemaphoreType.DMA,
        ],
    )
    def pool(x_hbm, tab_hbm, out_hbm, idx_v, rows_v, out_v, sem0, sem1):
        wid = lax.axis_index("s") * NC + lax.axis_index("c")
        base = wid * BPW
        sems = (sem0, sem1)

        pltpu.sync_copy(x_hbm.at[pl.ds(base, BPW)], idx_v)

        def gather_start(b, j, p):
            pltpu.make_async_copy(
                tab_hbm.at[idx_v.at[b, j]], rows_v.at[p], sems[p]
            ).start()

        def gather_wait(b, j, p):
            pltpu.make_async_copy(
                tab_hbm.at[idx_v.at[b, j]], rows_v.at[p], sems[p]
            ).wait()

        lanes = lax.iota(jnp.int32, 16)
        perms = tuple((lanes ^ k)[:, None] for k in (8, 4, 2, 1))
        def compute_half(p, j, acc):
            def step(it, acc):
                for u in range(GROUP):
                    l = it * GROUP + u
                    a0, a1, a2, a3 = acc
                    r0 = rows_v[p, l, pl.ds(0, 16)]
                    r1 = rows_v[p, l, pl.ds(16, 16)]
                    r2 = rows_v[p, l, pl.ds(32, 16)]
                    r3 = rows_v[p, l, pl.ds(48, 16)]
                    q = r0 * r0 + r1 * r1 + r2 * r2 + r3 * r3
                    ss = _lane_sum(q, perms)
                    y = _rsqrt_vec(ss)
                    factor = jnp.where(ss > jnp.float32(MAXN * MAXN), y, jnp.float32(1.0))
                    wl = jnp.float32(0.1) + (
                        jnp.float32(j * HALF) + l.astype(jnp.float32)
                    ) * jnp.float32(W_STEP)
                    cf = factor * wl
                    acc = (a0 + cf * r0, a1 + cf * r1, a2 + cf * r2, a3 + cf * r3)
                return acc

            return lax.fori_loop(0, HALF // GROUP, step, acc, unroll=False)

        def seq_body(b, _):
            gather_start(b, 1, 1)
            gather_wait(b, 0, 0)
            zero = jnp.zeros((16,), jnp.float32)
            acc = compute_half(0, 0, (zero, zero, zero, zero))

            @pl.when(b + 1 < BPW)
            def _():
                gather_start(b + 1, 0, 0)

            gather_wait(b, 1, 1)
            a0, a1, a2, a3 = compute_half(1, 1, acc)
            out_v[b, pl.ds(0, 16)] = a0
            out_v[b, pl.ds(16, 16)] = a1
            out_v[b, pl.ds(32, 16)] = a2
            out_v[b, pl.ds(48, 16)] = a3
            return 0

        gather_start(0, 0, 0)
        lax.fori_loop(0, BPW, seq_body, 0, unroll=False)
        pltpu.sync_copy(out_v, out_hbm.at[pl.ds(base, BPW)])

    return pool(x3, table)


def _mlp_tc(s_raw, W1, b1, W2, b2):
    def body(s_ref, w1_ref, b1_ref, w2_ref, b2_ref, o_ref):
        s = s_ref[...]
        ss = jnp.sum(s * s, axis=-1, keepdims=True)
        s = s * jax.lax.rsqrt(jnp.maximum(ss, jnp.float32(1e-24)))
        h = jnp.tanh(
            lax.dot_general(
                s, w1_ref[...], (((1,), (1,)), ((), ())),
                preferred_element_type=jnp.float32,
            )
            + b1_ref[...]
        )
        o = jnp.tanh(
            lax.dot_general(
                h, w2_ref[...], (((1,), (1,)), ((), ())),
                preferred_element_type=jnp.float32,
            )
            + b2_ref[...]
        )
        o_ref[...] = o

    return pl.pallas_call(
        body,
        out_shape=jax.ShapeDtypeStruct((B, D), jnp.float32),
    )(s_raw, W1, b1, W2, b2)


def kernel(x, table, W1, b1, W2, b2):
    x3 = x.astype(jnp.int32).reshape(B, 2, HALF)
    s_raw = _pool_sc(x3, table)
    return _mlp_tc(s_raw, W1, b1, W2, b2)
